# baseline (speedup 1.0000x reference)
import jax
import jax.numpy as jnp
from jax.experimental import pallas as pl
from jax.experimental.pallas import tpu as pltpu

_DIM = 2048
_LORA_DIMS = (8, 16, 32, 48, 64, 96, 128)
_NE = 7
_STACK = 512
_STARTS = (0, 8, 24, 56, 104, 168, 264)
_BOUNDS = (8, 24, 56, 104, 168, 264, 392)
_BLK = 1024

_EXP_PREC = jax.lax.Precision.DEFAULT


def _moe_body(x_ref, p_ref, *rest):
    wa_refs = rest[0:_NE]
    wb_refs = rest[_NE:2 * _NE]
    o_ref = rest[2 * _NE]
    a_s = rest[2 * _NE + 1]
    b_s = rest[2 * _NE + 2]
    seg_s = rest[2 * _NE + 3]
    p_s = rest[2 * _NE + 4]

    @pl.when(pl.program_id(0) == 0)
    def _assemble():
        a_s[...] = jnp.zeros_like(a_s)
        b_s[...] = jnp.zeros_like(b_s)
        for i in range(_NE):
            s, d = _STARTS[i], _LORA_DIMS[i]
            a_s[s:s + d, :] = wa_refs[i][...].astype(jnp.bfloat16)
            b_s[:, s:s + d] = wb_refs[i][...].astype(jnp.bfloat16)
        p_s[...] = p_ref[...].astype(jnp.bfloat16)
        segc = jax.lax.broadcasted_iota(jnp.int32, (8, _STACK), 1)
        seg = jnp.zeros((8, _STACK), jnp.int32)
        for b in _BOUNDS:
            seg += (segc >= b).astype(jnp.int32)
        seg_s[...] = seg

    half = _BLK // 8
    for sub in range(8):
        xb = x_ref[sub * half:(sub + 1) * half, :].astype(jnp.bfloat16)
        logits = jax.lax.dot_general(
            xb, p_s[...], (((1,), (1,)), ((), ())),
            preferred_element_type=jnp.float32,
            precision=jax.lax.Precision.DEFAULT)
        col8 = jax.lax.broadcasted_iota(jnp.int32, logits.shape, 1)
        logits = jnp.where(col8 < _NE, logits, jnp.float32(-3e38))
        m = jnp.max(logits, axis=1, keepdims=True)
        e = jnp.min(jnp.where(logits >= m, col8, _NE), axis=1, keepdims=True)

        h = jax.lax.dot_general(
            xb, a_s[...], (((1,), (1,)), ((), ())),
            preferred_element_type=jnp.float32, precision=_EXP_PREC)
        hm = jnp.where(seg_s[0:1, :] == e, h.astype(jnp.bfloat16),
                       jnp.bfloat16(0.0))

        o_ref[sub * half:(sub + 1) * half, :] = jax.lax.dot_general(
            hm, b_s[...], (((1,), (1,)), ((), ())),
            preferred_element_type=jnp.float32, precision=_EXP_PREC)


def kernel(x, prototypes, Wa0, Wa1, Wa2, Wa3, Wa4, Wa5, Wa6,
           Wb0, Wb1, Wb2, Wb3, Wb4, Wb5, Wb6):
    was = [Wa0, Wa1, Wa2, Wa3, Wa4, Wa5, Wa6]
    wbs = [Wb0, Wb1, Wb2, Wb3, Wb4, Wb5, Wb6]
    p_pad = jnp.pad(prototypes, ((0, 8 - _NE), (0, 0)))

    n_blk = x.shape[0] // _BLK
    const_spec = lambda shape: pl.BlockSpec(shape, lambda i: (0,) * len(shape))
    return pl.pallas_call(
        _moe_body,
        grid=(n_blk,),
        in_specs=[
            pl.BlockSpec((_BLK, _DIM), lambda i: (i, 0)),
            const_spec((8, _DIM)),
        ] + [const_spec((d, _DIM)) for d in _LORA_DIMS]
          + [const_spec((_DIM, d)) for d in _LORA_DIMS],
        out_specs=pl.BlockSpec((_BLK, _DIM), lambda i: (i, 0)),
        out_shape=jax.ShapeDtypeStruct((x.shape[0], _DIM), jnp.float32),
        scratch_shapes=[
            pltpu.VMEM((_STACK, _DIM), jnp.bfloat16),
            pltpu.VMEM((_DIM, _STACK), jnp.bfloat16),
            pltpu.VMEM((8, _STACK), jnp.int32),
            pltpu.VMEM((8, _DIM), jnp.bfloat16),
        ],
        compiler_params=pltpu.CompilerParams(
            dimension_semantics=("arbitrary",)),
    )(x, p_pad, *was, *wbs)


# speedup vs baseline: 1.6145x; 1.6145x over previous
import jax
import jax.numpy as jnp
from jax.experimental import pallas as pl
from jax.experimental.pallas import tpu as pltpu

_DIM = 2048
_LORA_DIMS = (8, 16, 32, 48, 64, 96, 128)
_NE = 7
_STACK = 512
_STARTS = (0, 8, 24, 56, 104, 168, 264)
_BOUNDS = (8, 24, 56, 104, 168, 264, 392)
_BLK = 1024

_EXP_PREC = jax.lax.Precision.DEFAULT


def _moe_body(x_ref, p_ref, *rest):
    wa_refs = rest[0:_NE]
    wb_refs = rest[_NE:2 * _NE]
    o_ref = rest[2 * _NE]
    a_s = rest[2 * _NE + 1]
    b_s = rest[2 * _NE + 2]
    seg_s = rest[2 * _NE + 3]
    p_s = rest[2 * _NE + 4]

    @pl.when(pl.program_id(0) == 0)
    def _assemble():
        a_s[...] = jnp.zeros_like(a_s)
        b_s[...] = jnp.zeros_like(b_s)
        for i in range(_NE):
            s, d = _STARTS[i], _LORA_DIMS[i]
            a_s[s:s + d, :] = wa_refs[i][...].astype(jnp.bfloat16)
            b_s[:, s:s + d] = wb_refs[i][...].astype(jnp.bfloat16)
        p_s[...] = p_ref[...].astype(jnp.bfloat16)
        segc = jax.lax.broadcasted_iota(jnp.int32, (8, _STACK), 1)
        seg = jnp.zeros((8, _STACK), jnp.int32)
        for b in _BOUNDS:
            seg += (segc >= b).astype(jnp.int32)
        seg_s[...] = seg

    half = _BLK // 4
    for sub in range(4):
        xb = x_ref[sub * half:(sub + 1) * half, :].astype(jnp.bfloat16)
        logits = jax.lax.dot_general(
            xb, p_s[...], (((1,), (1,)), ((), ())),
            preferred_element_type=jnp.float32,
            precision=jax.lax.Precision.DEFAULT)
        col8 = jax.lax.broadcasted_iota(jnp.int32, logits.shape, 1)
        logits = jnp.where(col8 < _NE, logits, jnp.float32(-3e38))
        m = jnp.max(logits, axis=1, keepdims=True)
        e = jnp.min(jnp.where(logits >= m, col8, _NE), axis=1, keepdims=True)

        h = jax.lax.dot_general(
            xb, a_s[...], (((1,), (1,)), ((), ())),
            preferred_element_type=jnp.float32, precision=_EXP_PREC)
        hm = jnp.where(seg_s[0:1, :] == e, h.astype(jnp.bfloat16),
                       jnp.bfloat16(0.0))

        o_ref[sub * half:(sub + 1) * half, :] = jax.lax.dot_general(
            hm, b_s[...], (((1,), (1,)), ((), ())),
            preferred_element_type=jnp.float32, precision=_EXP_PREC)


def kernel(x, prototypes, Wa0, Wa1, Wa2, Wa3, Wa4, Wa5, Wa6,
           Wb0, Wb1, Wb2, Wb3, Wb4, Wb5, Wb6):
    was = [Wa0, Wa1, Wa2, Wa3, Wa4, Wa5, Wa6]
    wbs = [Wb0, Wb1, Wb2, Wb3, Wb4, Wb5, Wb6]
    p_pad = jnp.pad(prototypes, ((0, 8 - _NE), (0, 0)))

    n_blk = x.shape[0] // _BLK
    const_spec = lambda shape: pl.BlockSpec(shape, lambda i: (0,) * len(shape))
    return pl.pallas_call(
        _moe_body,
        grid=(n_blk,),
        in_specs=[
            pl.BlockSpec((_BLK, _DIM), lambda i: (i, 0)),
            const_spec((8, _DIM)),
        ] + [const_spec((d, _DIM)) for d in _LORA_DIMS]
          + [const_spec((_DIM, d)) for d in _LORA_DIMS],
        out_specs=pl.BlockSpec((_BLK, _DIM), lambda i: (i, 0)),
        out_shape=jax.ShapeDtypeStruct((x.shape[0], _DIM), jnp.float32),
        scratch_shapes=[
            pltpu.VMEM((_STACK, _DIM), jnp.bfloat16),
            pltpu.VMEM((_DIM, _STACK), jnp.bfloat16),
            pltpu.VMEM((8, _STACK), jnp.int32),
            pltpu.VMEM((8, _DIM), jnp.bfloat16),
        ],
        compiler_params=pltpu.CompilerParams(
            dimension_semantics=("arbitrary",)),
    )(x, p_pad, *was, *wbs)
